# Initial kernel scaffold; baseline (speedup 1.0000x reference)
import jax
import jax.numpy as jnp
from jax.experimental import pallas as pl


def kernel(x, edge_index, pos_src, pos_dst, neg_src, neg_dst, ws1, wn1, b1, ws2, wn2, b2, ws3, wn3, b3, p1, pb1, p2, pb2, p3, pb3):
    raise NotImplementedError("write your pallas kernel here")


# speedup vs baseline: 3.7930x; 3.7930x over previous
import functools

import jax
import jax.numpy as jnp
from jax import lax
from jax.experimental import pallas as pl
from jax.experimental.pallas import tpu as pltpu
from jax.experimental.pallas import tpu_sc as plsc

NC = 2
NS = 16
NW = NC * NS


def _mesh():
  return plsc.VectorSubcoreMesh(core_axis_name="c", subcore_axis_name="s")


@functools.lru_cache(maxsize=None)
def _seg_sum_kernel(n, dc, e, k):
  per_w = e // NW
  nblk = per_w // k
  stripe = (-(-n // NS) + 7) // 8 * 8
  n_pad = NS * stripe
  chunks = []
  o = 0
  while o < stripe:
    sz = min(k, stripe - o)
    chunks.append((o, sz))
    o += sz

  @functools.partial(
      pl.kernel,
      out_type=jax.ShapeDtypeStruct((NC, n_pad, dc), jnp.float32),
      mesh=_mesh(),
      scratch_types=[
          pltpu.VMEM((k,), jnp.int32),
          pltpu.VMEM((k,), jnp.int32),
          pltpu.VMEM((k, dc), jnp.float32),
          pltpu.VMEM_SHARED((n_pad, dc), jnp.float32),
          pltpu.SemaphoreType.DMA,
      ],
  )
  def kern(table_hbm, src_hbm, dst_hbm, out_hbm, src_v, dst_v, rows_v, acc_sh, sem):
    cid = lax.axis_index("c")
    sid = lax.axis_index("s")
    wid = sid * NC + cid

    zvec = jnp.zeros((16,), jnp.float32)

    def zbody(r, carry):
      for j in range(dc // 16):
        rows_v[r, pl.ds(j * 16, 16)] = zvec
      return carry

    lax.fori_loop(0, k, zbody, 0)
    row0 = sid * stripe
    for (o, sz) in chunks:
      pltpu.sync_copy(rows_v.at[pl.ds(0, sz)], acc_sh.at[pl.ds(row0 + o, sz)])
    plsc.subcore_barrier()

    base = wid * per_w

    def ebody(i, carry):
      off = base + i * k
      pltpu.sync_copy(src_hbm.at[pl.ds(off, k)], src_v)
      pltpu.sync_copy(dst_hbm.at[pl.ds(off, k)], dst_v)
      pltpu.async_copy(table_hbm.at[src_v], rows_v, sem).wait()
      pltpu.sync_copy(rows_v, acc_sh.at[dst_v], add=True)
      return carry

    lax.fori_loop(0, nblk, ebody, 0)
    plsc.subcore_barrier()

    for (o, sz) in chunks:
      pltpu.sync_copy(acc_sh.at[pl.ds(row0 + o, sz)], rows_v.at[pl.ds(0, sz)])
      pltpu.sync_copy(rows_v.at[pl.ds(0, sz)], out_hbm.at[cid, pl.ds(row0 + o, sz)])

  return kern


@functools.lru_cache(maxsize=None)
def _deg_kernel(n, e, k, w):
  per_w = e // NW
  nblk = per_w // k
  stripe = (-(-n // NS) + 7) // 8 * 8
  n_pad = NS * stripe
  chunks = []
  o = 0
  while o < stripe:
    sz = min(k, stripe - o)
    chunks.append((o, sz))
    o += sz

  @functools.partial(
      pl.kernel,
      out_type=jax.ShapeDtypeStruct((NC, n_pad, w), jnp.float32),
      mesh=_mesh(),
      scratch_types=[
          pltpu.VMEM((k,), jnp.int32),
          pltpu.VMEM((k, w), jnp.float32),
          pltpu.VMEM_SHARED((n_pad, w), jnp.float32),
          pltpu.SemaphoreType.DMA,
      ],
  )
  def kern(src_hbm, dst_hbm, out_hbm, dst_v, rows_v, acc_sh, sem):
    del src_hbm, sem
    cid = lax.axis_index("c")
    sid = lax.axis_index("s")
    wid = sid * NC + cid

    def fill(val):
      vec = jnp.full((16,), val, jnp.float32)

      def body(r, carry):
        for j in range(w // 16):
          rows_v[r, pl.ds(j * 16, 16)] = vec
        return carry

      lax.fori_loop(0, k, body, 0)

    fill(0.0)
    row0 = sid * stripe
    for (o, sz) in chunks:
      pltpu.sync_copy(rows_v.at[pl.ds(0, sz)], acc_sh.at[pl.ds(row0 + o, sz)])
    plsc.subcore_barrier()

    fill(1.0)
    base = wid * per_w

    def ebody(i, carry):
      off = base + i * k
      pltpu.sync_copy(dst_hbm.at[pl.ds(off, k)], dst_v)
      pltpu.sync_copy(rows_v, acc_sh.at[dst_v], add=True)
      return carry

    lax.fori_loop(0, nblk, ebody, 0)
    plsc.subcore_barrier()

    for (o, sz) in chunks:
      pltpu.sync_copy(acc_sh.at[pl.ds(row0 + o, sz)], rows_v.at[pl.ds(0, sz)])
      pltpu.sync_copy(rows_v.at[pl.ds(0, sz)], out_hbm.at[cid, pl.ds(row0 + o, sz)])

  return kern


@functools.lru_cache(maxsize=None)
def _gather_kernel(nidx, d):
  per_w = nidx // NW

  @functools.partial(
      pl.kernel,
      out_type=jax.ShapeDtypeStruct((nidx, d), jnp.float32),
      mesh=_mesh(),
      scratch_types=[
          pltpu.VMEM((per_w,), jnp.int32),
          pltpu.VMEM((per_w, d), jnp.float32),
          pltpu.SemaphoreType.DMA,
      ],
  )
  def kern(table_hbm, idx_hbm, out_hbm, idx_v, rows_v, sem):
    cid = lax.axis_index("c")
    sid = lax.axis_index("s")
    wid = sid * NC + cid
    base = wid * per_w
    pltpu.sync_copy(idx_hbm.at[pl.ds(base, per_w)], idx_v)
    pltpu.async_copy(table_hbm.at[idx_v], rows_v, sem).wait()
    pltpu.sync_copy(rows_v, out_hbm.at[pl.ds(base, per_w)])

  return kern


@functools.lru_cache(maxsize=None)
def _layer_fn(n, din, dout, nchunks, dc, relu, rblk, wdeg):
  grid = n // rblk

  def body(*refs):
    h_ref = refs[0]
    agg_refs = refs[1:1 + nchunks]
    deg_ref = refs[1 + nchunks]
    ws_ref, wn_ref, b_ref, out_ref = refs[2 + nchunks:]
    deg = deg_ref[0, :, 0:1] + deg_ref[1, :, 0:1]
    rd = 1.0 / jnp.maximum(deg, 1.0)
    mean = jnp.concatenate([a[0] + a[1] for a in agg_refs], axis=1) * rd
    acc = jnp.dot(h_ref[...], ws_ref[...], preferred_element_type=jnp.float32)
    acc = acc + jnp.dot(mean, wn_ref[...], preferred_element_type=jnp.float32)
    acc = acc + b_ref[...]
    if relu:
      acc = jnp.maximum(acc, 0.0)
    out_ref[...] = acc

  in_specs = [pl.BlockSpec((rblk, din), lambda i: (i, 0))]
  in_specs += [pl.BlockSpec((NC, rblk, dc), lambda i: (0, i, 0))] * nchunks
  in_specs += [pl.BlockSpec((NC, rblk, wdeg), lambda i: (0, i, 0))]
  in_specs += [
      pl.BlockSpec((din, dout), lambda i: (0, 0)),
      pl.BlockSpec((din, dout), lambda i: (0, 0)),
      pl.BlockSpec((1, dout), lambda i: (0, 0)),
  ]
  return pl.pallas_call(
      body,
      grid=(grid,),
      in_specs=in_specs,
      out_specs=pl.BlockSpec((rblk, dout), lambda i: (i, 0)),
      out_shape=jax.ShapeDtypeStruct((n, dout), jnp.float32),
  )


@functools.lru_cache(maxsize=None)
def _predict_fn(nrows, d):
  b = nrows // 2

  def body(r_ref, p1_ref, b1_ref, p2_ref, b2_ref, p3_ref, b3_ref, out_ref):
    hs = r_ref[0:b, :]
    hd = r_ref[b:2 * b, :]
    z = hs * hd
    z = jnp.dot(z, p1_ref[...], preferred_element_type=jnp.float32) + b1_ref[...]
    z = jnp.maximum(z, 0.0)
    z = jnp.dot(z, p2_ref[...], preferred_element_type=jnp.float32) + b2_ref[...]
    z = jnp.maximum(z, 0.0)
    out_ref[...] = (
        jnp.dot(z, p3_ref[...], preferred_element_type=jnp.float32) + b3_ref[...]
    )

  return pl.pallas_call(
      body,
      out_shape=jax.ShapeDtypeStruct((b, 128), jnp.float32),
  )


def kernel(x, edge_index, pos_src, pos_dst, neg_src, neg_dst,
           ws1, wn1, b1, ws2, wn2, b2, ws3, wn3, b3,
           p1, pb1, p2, pb2, p3, pb3):
  n, d_in = x.shape
  e = edge_index.shape[1]
  d_h = ws1.shape[1]
  src = edge_index[0].astype(jnp.int32)
  dst = edge_index[1].astype(jnp.int32)
  k = 200
  dc = 128
  rblk = 1000

  wdeg = 128
  degp = _deg_kernel(n, e, k, wdeg)(src, dst)

  def sage(h, ws, wn, b, relu):
    din = h.shape[1]
    nchunks = din // dc
    aggs = [
        _seg_sum_kernel(n, dc, e, k)(h[:, c * dc:(c + 1) * dc], src, dst)
        for c in range(nchunks)
    ]
    f = _layer_fn(n, din, d_h, nchunks, dc, relu, rblk, wdeg)
    return f(h, *aggs, degp, ws, wn, b.reshape(1, -1))

  h = sage(x, ws1, wn1, b1, True)
  h = sage(h, ws2, wn2, b2, True)
  h = sage(h, ws3, wn3, b3, False)

  all_idx = jnp.concatenate(
      [pos_src, neg_src, pos_dst, neg_dst]).astype(jnp.int32)
  rows = _gather_kernel(all_idx.shape[0], d_h)(h, all_idx)

  p3p = jnp.pad(p3, ((0, 0), (0, 127)))
  pb3p = jnp.pad(pb3, ((0, 127))).reshape(1, -1)
  scores = _predict_fn(rows.shape[0], d_h)(
      rows, p1, pb1.reshape(1, -1), p2, pb2.reshape(1, -1), p3p, pb3p)
  bsz = scores.shape[0] // 2
  h_pos = scores[:bsz, 0:1]
  h_neg = scores[bsz:, 0:1]
  return (h_pos, h_neg)


# speedup vs baseline: 3.8251x; 1.0085x over previous
import functools

import jax
import jax.numpy as jnp
from jax import lax
from jax.experimental import pallas as pl
from jax.experimental.pallas import tpu as pltpu
from jax.experimental.pallas import tpu_sc as plsc

NC = 2
NS = 16
NW = NC * NS


def _mesh():
  return plsc.VectorSubcoreMesh(core_axis_name="c", subcore_axis_name="s")


@functools.lru_cache(maxsize=None)
def _seg_sum_kernel(n, dc, e, k):
  per_w = e // NW
  nblk = per_w // k
  stripe = (-(-n // NS) + 7) // 8 * 8
  n_pad = NS * stripe
  chunks = []
  o = 0
  while o < stripe:
    sz = min(k, stripe - o)
    chunks.append((o, sz))
    o += sz

  tail = per_w - nblk * k
  assert nblk % 2 == 1 and nblk >= 3 and tail % 8 == 0
  npairs = (nblk - 1) // 2
  tail_s = max(tail, 8)

  @functools.partial(
      pl.kernel,
      out_type=jax.ShapeDtypeStruct((NC, n_pad, dc), jnp.float32),
      mesh=_mesh(),
      scratch_types=[
          pltpu.VMEM((k,), jnp.int32),
          pltpu.VMEM((k,), jnp.int32),
          pltpu.VMEM((k,), jnp.int32),
          pltpu.VMEM((k,), jnp.int32),
          pltpu.VMEM((tail_s,), jnp.int32),
          pltpu.VMEM((tail_s,), jnp.int32),
          pltpu.VMEM((k, dc), jnp.float32),
          pltpu.VMEM((k, dc), jnp.float32),
          pltpu.VMEM((tail_s, dc), jnp.float32),
          pltpu.VMEM_SHARED((n_pad, dc), jnp.float32),
          pltpu.SemaphoreType.DMA,
          pltpu.SemaphoreType.DMA,
      ],
  )
  def kern(table_hbm, src_hbm, dst_hbm, out_hbm,
           src0, dst0, src1, dst1, src_t, dst_t, rows0, rows1, rows_t,
           acc_sh, sem0, sem1):
    cid = lax.axis_index("c")
    sid = lax.axis_index("s")
    wid = sid * NC + cid

    zvec = jnp.zeros((16,), jnp.float32)

    def zbody(r, carry):
      for j in range(dc // 16):
        rows0[r, pl.ds(j * 16, 16)] = zvec
      return carry

    lax.fori_loop(0, k, zbody, 0)
    row0 = sid * stripe
    for (o, sz) in chunks:
      pltpu.sync_copy(rows0.at[pl.ds(0, sz)], acc_sh.at[pl.ds(row0 + o, sz)])
    plsc.subcore_barrier()

    base = wid * per_w

    def fetch(off, bs, src_v, dst_v, rows_v, sem):
      pltpu.sync_copy(src_hbm.at[pl.ds(off, bs)], src_v)
      pltpu.sync_copy(dst_hbm.at[pl.ds(off, bs)], dst_v)
      return pltpu.async_copy(table_hbm.at[src_v], rows_v, sem)

    def scatter(dst_v, rows_v):
      pltpu.sync_copy(rows_v, acc_sh.at[dst_v], add=True)

    fetch(base, k, src0, dst0, rows0, sem0).wait()

    def pbody(p, carry):
      i = 2 * p
      d1 = fetch(base + (i + 1) * k, k, src1, dst1, rows1, sem1)
      scatter(dst0, rows0)
      d1.wait()
      d0 = fetch(base + (i + 2) * k, k, src0, dst0, rows0, sem0)
      scatter(dst1, rows1)
      d0.wait()
      return carry

    lax.fori_loop(0, npairs, pbody, 0)
    if tail:
      dt = fetch(base + nblk * k, tail, src_t, dst_t, rows_t, sem1)
      scatter(dst0, rows0)
      dt.wait()
      scatter(dst_t, rows_t)
    else:
      scatter(dst0, rows0)
    plsc.subcore_barrier()

    for (o, sz) in chunks:
      pltpu.sync_copy(acc_sh.at[pl.ds(row0 + o, sz)], rows0.at[pl.ds(0, sz)])
      pltpu.sync_copy(rows0.at[pl.ds(0, sz)], out_hbm.at[cid, pl.ds(row0 + o, sz)])

  return kern


@functools.lru_cache(maxsize=None)
def _deg_kernel(n, e, k, w):
  per_w = e // NW
  nblk = per_w // k
  stripe = (-(-n // NS) + 7) // 8 * 8
  n_pad = NS * stripe
  chunks = []
  o = 0
  while o < stripe:
    sz = min(k, stripe - o)
    chunks.append((o, sz))
    o += sz

  @functools.partial(
      pl.kernel,
      out_type=jax.ShapeDtypeStruct((NC, n_pad, w), jnp.float32),
      mesh=_mesh(),
      scratch_types=[
          pltpu.VMEM((k,), jnp.int32),
          pltpu.VMEM((k, w), jnp.float32),
          pltpu.VMEM_SHARED((n_pad, w), jnp.float32),
          pltpu.SemaphoreType.DMA,
      ],
  )
  def kern(src_hbm, dst_hbm, out_hbm, dst_v, rows_v, acc_sh, sem):
    del src_hbm, sem
    cid = lax.axis_index("c")
    sid = lax.axis_index("s")
    wid = sid * NC + cid

    def fill(val):
      vec = jnp.full((16,), val, jnp.float32)

      def body(r, carry):
        for j in range(w // 16):
          rows_v[r, pl.ds(j * 16, 16)] = vec
        return carry

      lax.fori_loop(0, k, body, 0)

    fill(0.0)
    row0 = sid * stripe
    for (o, sz) in chunks:
      pltpu.sync_copy(rows_v.at[pl.ds(0, sz)], acc_sh.at[pl.ds(row0 + o, sz)])
    plsc.subcore_barrier()

    fill(1.0)
    base = wid * per_w

    def ebody(i, carry):
      off = base + i * k
      pltpu.sync_copy(dst_hbm.at[pl.ds(off, k)], dst_v)
      pltpu.sync_copy(rows_v, acc_sh.at[dst_v], add=True)
      return carry

    lax.fori_loop(0, nblk, ebody, 0)
    plsc.subcore_barrier()

    for (o, sz) in chunks:
      pltpu.sync_copy(acc_sh.at[pl.ds(row0 + o, sz)], rows_v.at[pl.ds(0, sz)])
      pltpu.sync_copy(rows_v.at[pl.ds(0, sz)], out_hbm.at[cid, pl.ds(row0 + o, sz)])

  return kern


@functools.lru_cache(maxsize=None)
def _gather_kernel(nidx, d):
  per_w = nidx // NW

  @functools.partial(
      pl.kernel,
      out_type=jax.ShapeDtypeStruct((nidx, d), jnp.float32),
      mesh=_mesh(),
      scratch_types=[
          pltpu.VMEM((per_w,), jnp.int32),
          pltpu.VMEM((per_w, d), jnp.float32),
          pltpu.SemaphoreType.DMA,
      ],
  )
  def kern(table_hbm, idx_hbm, out_hbm, idx_v, rows_v, sem):
    cid = lax.axis_index("c")
    sid = lax.axis_index("s")
    wid = sid * NC + cid
    base = wid * per_w
    pltpu.sync_copy(idx_hbm.at[pl.ds(base, per_w)], idx_v)
    pltpu.async_copy(table_hbm.at[idx_v], rows_v, sem).wait()
    pltpu.sync_copy(rows_v, out_hbm.at[pl.ds(base, per_w)])

  return kern


@functools.lru_cache(maxsize=None)
def _layer_fn(n, din, dout, nchunks, dc, relu, rblk, wdeg):
  grid = n // rblk

  def body(*refs):
    h_ref = refs[0]
    agg_refs = refs[1:1 + nchunks]
    deg_ref = refs[1 + nchunks]
    ws_ref, wn_ref, b_ref, out_ref = refs[2 + nchunks:]
    deg = deg_ref[0, :, 0:1] + deg_ref[1, :, 0:1]
    rd = 1.0 / jnp.maximum(deg, 1.0)
    mean = jnp.concatenate([a[0] + a[1] for a in agg_refs], axis=1) * rd
    acc = jnp.dot(h_ref[...], ws_ref[...], preferred_element_type=jnp.float32)
    acc = acc + jnp.dot(mean, wn_ref[...], preferred_element_type=jnp.float32)
    acc = acc + b_ref[...]
    if relu:
      acc = jnp.maximum(acc, 0.0)
    out_ref[...] = acc

  in_specs = [pl.BlockSpec((rblk, din), lambda i: (i, 0))]
  in_specs += [pl.BlockSpec((NC, rblk, dc), lambda i: (0, i, 0))] * nchunks
  in_specs += [pl.BlockSpec((NC, rblk, wdeg), lambda i: (0, i, 0))]
  in_specs += [
      pl.BlockSpec((din, dout), lambda i: (0, 0)),
      pl.BlockSpec((din, dout), lambda i: (0, 0)),
      pl.BlockSpec((1, dout), lambda i: (0, 0)),
  ]
  return pl.pallas_call(
      body,
      grid=(grid,),
      in_specs=in_specs,
      out_specs=pl.BlockSpec((rblk, dout), lambda i: (i, 0)),
      out_shape=jax.ShapeDtypeStruct((n, dout), jnp.float32),
  )


@functools.lru_cache(maxsize=None)
def _predict_fn(nrows, d):
  b = nrows // 2

  def body(r_ref, p1_ref, b1_ref, p2_ref, b2_ref, p3_ref, b3_ref, out_ref):
    hs = r_ref[0:b, :]
    hd = r_ref[b:2 * b, :]
    z = hs * hd
    z = jnp.dot(z, p1_ref[...], preferred_element_type=jnp.float32) + b1_ref[...]
    z = jnp.maximum(z, 0.0)
    z = jnp.dot(z, p2_ref[...], preferred_element_type=jnp.float32) + b2_ref[...]
    z = jnp.maximum(z, 0.0)
    out_ref[...] = (
        jnp.dot(z, p3_ref[...], preferred_element_type=jnp.float32) + b3_ref[...]
    )

  return pl.pallas_call(
      body,
      out_shape=jax.ShapeDtypeStruct((b, 128), jnp.float32),
  )


def kernel(x, edge_index, pos_src, pos_dst, neg_src, neg_dst,
           ws1, wn1, b1, ws2, wn2, b2, ws3, wn3, b3,
           p1, pb1, p2, pb2, p3, pb3):
  n, d_in = x.shape
  e = edge_index.shape[1]
  d_h = ws1.shape[1]
  src = edge_index[0].astype(jnp.int32)
  dst = edge_index[1].astype(jnp.int32)
  k = 128
  k_deg = 200
  dc = 128
  rblk = 1000

  wdeg = 128
  degp = _deg_kernel(n, e, k_deg, wdeg)(src, dst)

  def sage(h, ws, wn, b, relu):
    din = h.shape[1]
    nchunks = din // dc
    aggs = [
        _seg_sum_kernel(n, dc, e, k)(h[:, c * dc:(c + 1) * dc], src, dst)
        for c in range(nchunks)
    ]
    f = _layer_fn(n, din, d_h, nchunks, dc, relu, rblk, wdeg)
    return f(h, *aggs, degp, ws, wn, b.reshape(1, -1))

  h = sage(x, ws1, wn1, b1, True)
  h = sage(h, ws2, wn2, b2, True)
  h = sage(h, ws3, wn3, b3, False)

  all_idx = jnp.concatenate(
      [pos_src, neg_src, pos_dst, neg_dst]).astype(jnp.int32)
  rows = _gather_kernel(all_idx.shape[0], d_h)(h, all_idx)

  p3p = jnp.pad(p3, ((0, 0), (0, 127)))
  pb3p = jnp.pad(pb3, ((0, 127))).reshape(1, -1)
  scores = _predict_fn(rows.shape[0], d_h)(
      rows, p1, pb1.reshape(1, -1), p2, pb2.reshape(1, -1), p3p, pb3p)
  bsz = scores.shape[0] // 2
  h_pos = scores[:bsz, 0:1]
  h_neg = scores[bsz:, 0:1]
  return (h_pos, h_neg)
